# baseline (speedup 1.0000x reference)
import functools

import jax
import jax.numpy as jnp
from jax import lax
from jax.experimental import pallas as pl
from jax.experimental.pallas import tpu as pltpu
from jax.experimental.pallas import tpu_sc as plsc

N = 10000
E = 320000
D = 128
H = 128
L = 4
G = 64
EPS = 0.1

NC = 2
NS = 16
LANES = 16
NW = NC * NS
EDGES_PER_TILE = E // NW
CHUNK = 80
NCHUNK = EDGES_PER_TILE // CHUNK
ROWS_PER_TILE = N // NS
ZROWS = 200

_MESH = plsc.VectorSubcoreMesh(core_axis_name="c", subcore_axis_name="s")
_SC_PARAMS = pltpu.CompilerParams(needs_layout_passes=False)


def _tanh(s):
  u = jnp.exp(-2.0 * jnp.abs(s))
  return jnp.sign(s) * (1.0 - u) / (1.0 + u)


@functools.partial(
    pl.kernel,
    out_type=jax.ShapeDtypeStruct((NC * N,), jnp.float32),
    mesh=_MESH,
    compiler_params=_SC_PARAMS,
    scratch_types=[
        pltpu.VMEM((CHUNK,), jnp.int32),
        pltpu.VMEM((CHUNK,), jnp.float32),
        pltpu.VMEM((N,), jnp.float32),
        pltpu.VMEM_SHARED((N,), jnp.float32),
    ],
)
def _sc_degree(dst_hbm, deg_hbm, dst_v, ones_v, zbuf_v, deg_sh):
  cid = lax.axis_index("c")
  sid = lax.axis_index("s")
  wid = cid * NS + sid

  def _zero(i, _):
    zbuf_v[pl.ds(i * LANES, LANES)] = jnp.zeros((LANES,), jnp.float32)
    return 0

  def _ones(i, _):
    ones_v[pl.ds(i * LANES, LANES)] = jnp.ones((LANES,), jnp.float32)
    return 0

  lax.fori_loop(0, CHUNK // LANES, _ones, 0)

  @pl.when(sid == 0)
  def _():
    lax.fori_loop(0, N // LANES, _zero, 0)
    pltpu.sync_copy(zbuf_v, deg_sh)

  plsc.subcore_barrier()

  base = wid * EDGES_PER_TILE

  def _chunk(c, _):
    pltpu.sync_copy(dst_hbm.at[pl.ds(base + c * CHUNK, CHUNK)], dst_v)
    pltpu.sync_copy(ones_v, deg_sh.at[dst_v], add=True)
    return 0

  lax.fori_loop(0, NCHUNK, _chunk, 0)

  plsc.subcore_barrier()

  @pl.when(sid == 0)
  def _():
    pltpu.sync_copy(deg_sh, zbuf_v)
    pltpu.sync_copy(zbuf_v, deg_hbm.at[pl.ds(cid * N, N)])


@functools.partial(
    pl.kernel,
    out_type=jax.ShapeDtypeStruct((NC, N, H), jnp.float32),
    mesh=_MESH,
    compiler_params=_SC_PARAMS,
    scratch_types=[
        [pltpu.VMEM((2 * CHUNK,), jnp.int32)] * 2,
        [pltpu.VMEM((CHUNK,), jnp.float32)] * 2,
        [pltpu.VMEM((CHUNK,), jnp.float32)] * 2,
        [pltpu.VMEM((CHUNK,), jnp.int32)] * 2,
        pltpu.VMEM((CHUNK,), jnp.float32),
        [pltpu.VMEM((CHUNK, H), jnp.float32)] * 2,
        [pltpu.VMEM((CHUNK, H), jnp.float32)] * 2,
        [pltpu.SemaphoreType.DMA] * 2,
        [pltpu.SemaphoreType.DMA] * 2,
        pltpu.VMEM_SHARED((N, H), jnp.float32),
    ],
)
def _sc_layer(g_hbm, al_hbm, ar_hbm, eidx_hbm, agg_hbm,
              ev, alv, arv, dvs, coef_v, ra, rs, semg, sems, agg_sh):
  cid = lax.axis_index("c")
  sid = lax.axis_index("s")
  wid = cid * NS + sid
  base = wid * NCHUNK

  def _zero(i, _):
    ra[0][i // (H // LANES), pl.ds((i % (H // LANES)) * LANES, LANES)] = (
        jnp.zeros((LANES,), jnp.float32))
    return 0

  lax.fori_loop(0, CHUNK * (H // LANES), _zero, 0)

  @pl.when(sid == 0)
  def _():
    def _zcopy(k, _):
      pltpu.sync_copy(ra[0], agg_sh.at[pl.ds(k * CHUNK, CHUNK)])
      return 0

    lax.fori_loop(0, N // CHUNK, _zcopy, 0)

  plsc.subcore_barrier()

  def _fetch(c, b):
    pltpu.sync_copy(eidx_hbm.at[pl.ds((base + c) * 2 * CHUNK, 2 * CHUNK)],
                    ev[b])
    pltpu.async_copy(g_hbm.at[ev[b].at[pl.ds(0, CHUNK)]], ra[b], semg[b])
    pltpu.async_copy(al_hbm.at[ev[b].at[pl.ds(0, CHUNK)]], alv[b], semg[b])
    pltpu.async_copy(ar_hbm.at[ev[b].at[pl.ds(CHUNK, CHUNK)]], arv[b],
                     semg[b])

  def _step(c, b, prefetch):
    pltpu.make_async_copy(g_hbm.at[pl.ds(0, CHUNK)], ra[b], semg[b]).wait()
    pltpu.make_async_copy(al_hbm.at[pl.ds(0, CHUNK)], alv[b], semg[b]).wait()
    pltpu.make_async_copy(al_hbm.at[pl.ds(0, CHUNK)], arv[b], semg[b]).wait()

    @pl.when(c >= 2)
    def _():
      pltpu.make_async_copy(g_hbm.at[pl.ds(0, CHUNK)], rs[b], sems[b]).wait()

    def _coef(g, _):
      sl = pl.ds(g * LANES, LANES)
      coef_v[sl] = _tanh(alv[b][sl] + arv[b][sl])
      dvs[b][sl] = ev[b][pl.ds(CHUNK + g * LANES, LANES)]
      return 0

    lax.fori_loop(0, CHUNK // LANES, _coef, 0)

    def _scale(e, _):
      w = plsc.load_gather(coef_v, [jnp.full((LANES,), e, jnp.int32)])
      for j in range(H // LANES):
        sl = pl.ds(j * LANES, LANES)
        rs[b][e, sl] = ra[b][e, sl] * w
      return 0

    lax.fori_loop(0, CHUNK, _scale, 0)

    pltpu.async_copy(rs[b], agg_sh.at[dvs[b]], sems[b], add=True)

    if prefetch:
      @pl.when(c + 2 < NCHUNK)
      def _():
        _fetch(c + 2, b)

  _fetch(0, 0)
  _fetch(1, 1)

  def _pair(k, _):
    _step(2 * k, 0, True)
    _step(2 * k + 1, 1, True)
    return 0

  lax.fori_loop(0, NCHUNK // 2, _pair, 0)
  _step(NCHUNK - 1, 0, False)

  pltpu.make_async_copy(g_hbm.at[pl.ds(0, CHUNK)], rs[1], sems[1]).wait()
  pltpu.make_async_copy(g_hbm.at[pl.ds(0, CHUNK)], rs[0], sems[0]).wait()

  plsc.subcore_barrier()

  @pl.when(sid == 0)
  def _():
    pltpu.sync_copy(agg_sh, agg_hbm.at[cid])


_BLK = 1000
_NBLK = N // _BLK
_PREC = jax.lax.Precision.HIGHEST


def _tc_prolog_body(x_ref, w1_ref, b1_ref, deg_ref, watt_ref, batt_ref,
                    h_ref, g_ref, alr_ref, dis_ref):
  h = lax.dot_general(x_ref[...], w1_ref[...], (((1,), (1,)), ((), ())),
                      precision=_PREC) + b1_ref[...]
  h = jnp.maximum(h, 0.0)
  h_ref[...] = h
  alr_ref[...] = lax.dot_general(h, watt_ref[...], (((1,), (0,)), ((), ())),
                                 precision=_PREC) + batt_ref[...]
  deg = deg_ref[...]
  d = deg[:, 0:1] + deg[:, 1:2]
  dis = jnp.where(d > 0.0, lax.rsqrt(jnp.where(d > 0.0, d, 1.0)), 0.0)
  dis_ref[...] = dis
  g_ref[...] = h * dis


def _tc_prolog(x, t1_w, b1, deg2t, watt, batt):
  return pl.pallas_call(
      _tc_prolog_body,
      grid=(_NBLK,),
      in_specs=[
          pl.BlockSpec((_BLK, D), lambda i: (i, 0)),
          pl.BlockSpec((H, D), lambda i: (0, 0)),
          pl.BlockSpec((1, H), lambda i: (0, 0)),
          pl.BlockSpec((_BLK, 2), lambda i: (i, 0)),
          pl.BlockSpec((H, 2), lambda i: (0, 0)),
          pl.BlockSpec((1, 2), lambda i: (0, 0)),
      ],
      out_specs=[
          pl.BlockSpec((_BLK, H), lambda i: (i, 0)),
          pl.BlockSpec((_BLK, H), lambda i: (i, 0)),
          pl.BlockSpec((_BLK, 2), lambda i: (i, 0)),
          pl.BlockSpec((_BLK, 1), lambda i: (i, 0)),
      ],
      out_shape=[
          jax.ShapeDtypeStruct((N, H), jnp.float32),
          jax.ShapeDtypeStruct((N, H), jnp.float32),
          jax.ShapeDtypeStruct((N, 2), jnp.float32),
          jax.ShapeDtypeStruct((N, 1), jnp.float32),
      ],
  )(x, t1_w, b1, deg2t, watt, batt)


def _tc_combine_body(agg_ref, raw_ref, dis_ref, watt_ref, batt_ref,
                     g_ref, alr_ref):
  dis = dis_ref[...]
  h = dis * (agg_ref[0] + agg_ref[1]) + EPS * raw_ref[...]
  g_ref[...] = h * dis
  alr_ref[...] = lax.dot_general(h, watt_ref[...], (((1,), (0,)), ((), ())),
                                 precision=_PREC) + batt_ref[...]


def _tc_combine(aggp, raw, dis2, watt, batt):
  return pl.pallas_call(
      _tc_combine_body,
      grid=(_NBLK,),
      in_specs=[
          pl.BlockSpec((NC, _BLK, H), lambda i: (0, i, 0)),
          pl.BlockSpec((_BLK, H), lambda i: (i, 0)),
          pl.BlockSpec((_BLK, 1), lambda i: (i, 0)),
          pl.BlockSpec((H, 2), lambda i: (0, 0)),
          pl.BlockSpec((1, 2), lambda i: (0, 0)),
      ],
      out_specs=[
          pl.BlockSpec((_BLK, H), lambda i: (i, 0)),
          pl.BlockSpec((_BLK, 2), lambda i: (i, 0)),
      ],
      out_shape=[
          jax.ShapeDtypeStruct((N, H), jnp.float32),
          jax.ShapeDtypeStruct((N, 2), jnp.float32),
      ],
  )(aggp, raw, dis2, watt, batt)


def _tc_epilog_body(agg_ref, raw_ref, dis_ref, w2_ref, b2_ref, batch_ref,
                    h_ref, gemb_ref):
  i = pl.program_id(0)
  h = dis_ref[...] * (agg_ref[0] + agg_ref[1]) + EPS * raw_ref[...]
  oh = lax.dot_general(h, w2_ref[...], (((1,), (1,)), ((), ())),
                       precision=_PREC) + b2_ref[...]
  h_ref[...] = oh
  gids = lax.broadcasted_iota(jnp.int32, (1, G), 1)
  onehot = (batch_ref[...] == gids).astype(jnp.float32)
  contrib = lax.dot_general(onehot, oh, (((0,), (0,)), ((), ())),
                            precision=_PREC)

  @pl.when(i == 0)
  def _():
    gemb_ref[...] = jnp.zeros_like(gemb_ref)

  gemb_ref[...] += contrib


def _tc_epilog(aggp, raw, dis2, t2_w, b2, batch2):
  return pl.pallas_call(
      _tc_epilog_body,
      grid=(_NBLK,),
      in_specs=[
          pl.BlockSpec((NC, _BLK, H), lambda i: (0, i, 0)),
          pl.BlockSpec((_BLK, H), lambda i: (i, 0)),
          pl.BlockSpec((_BLK, 1), lambda i: (i, 0)),
          pl.BlockSpec((H, H), lambda i: (0, 0)),
          pl.BlockSpec((1, H), lambda i: (0, 0)),
          pl.BlockSpec((_BLK, 1), lambda i: (i, 0)),
      ],
      out_specs=[
          pl.BlockSpec((_BLK, H), lambda i: (i, 0)),
          pl.BlockSpec((G, H), lambda i: (0, 0)),
      ],
      out_shape=[
          jax.ShapeDtypeStruct((N, H), jnp.float32),
          jax.ShapeDtypeStruct((G, H), jnp.float32),
      ],
  )(aggp, raw, dis2, t2_w, b2, batch2)


def kernel(x, edge_index, batch, t1_w, t1_b, t2_w, t2_b,
           att_l_w, att_l_b, att_r_w, att_r_b):
  src = edge_index[0]
  dst = edge_index[1]

  watts = [jnp.stack([att_l_w[l], att_r_w[l]], axis=1) for l in range(L)]
  batts = [jnp.stack([att_l_b[l], att_r_b[l]]).reshape(1, 2) for l in range(L)]
  b1 = t1_b.reshape(1, H)
  b2 = t2_b.reshape(1, H)
  batch2 = batch.reshape(N, 1)
  eidx = jnp.concatenate(
      [src.reshape(-1, CHUNK), dst.reshape(-1, CHUNK)], axis=1).reshape(-1)

  degp = _sc_degree(dst).reshape(NC, N)
  raw, g, alr, dis2 = _tc_prolog(x, t1_w, b1, degp.T, watts[0], batts[0])

  for l in range(L):
    aggp = _sc_layer(g, alr[:, 0], alr[:, 1], eidx)
    if l < L - 1:
      g, alr = _tc_combine(aggp, raw, dis2, watts[l + 1], batts[l + 1])

  out_h, graph_emb = _tc_epilog(aggp, raw, dis2, t2_w, b2, batch2)
  return (graph_emb, out_h)


# speedup vs baseline: 2.0967x; 2.0967x over previous
import functools

import jax
import jax.numpy as jnp
from jax import lax
from jax.experimental import pallas as pl
from jax.experimental.pallas import tpu as pltpu
from jax.experimental.pallas import tpu_sc as plsc

N = 10000
E = 320000
D = 128
H = 128
L = 4
G = 64
EPS = 0.1

NC = 2
NS = 16
LANES = 16
NW = NC * NS
EDGES_PER_TILE = E // NW
CHUNK = 80
NCHUNK = EDGES_PER_TILE // CHUNK
ROWS_PER_TILE = N // NS
ZROWS = 200

_MESH = plsc.VectorSubcoreMesh(core_axis_name="c", subcore_axis_name="s")
_SC_PARAMS = pltpu.CompilerParams(needs_layout_passes=False)


def _tanh(s):
  u = jnp.exp(-2.0 * jnp.abs(s))
  return jnp.sign(s) * (1.0 - u) / (1.0 + u)


@functools.partial(
    pl.kernel,
    out_type=jax.ShapeDtypeStruct((NC * N,), jnp.float32),
    mesh=_MESH,
    compiler_params=_SC_PARAMS,
    scratch_types=[
        pltpu.VMEM((CHUNK,), jnp.int32),
        pltpu.VMEM((CHUNK,), jnp.float32),
        pltpu.VMEM((N,), jnp.float32),
        pltpu.VMEM_SHARED((N,), jnp.float32),
    ],
)
def _sc_degree(dst_hbm, deg_hbm, dst_v, ones_v, zbuf_v, deg_sh):
  cid = lax.axis_index("c")
  sid = lax.axis_index("s")
  wid = cid * NS + sid

  def _zero(i, _):
    zbuf_v[pl.ds(i * LANES, LANES)] = jnp.zeros((LANES,), jnp.float32)
    return 0

  def _ones(i, _):
    ones_v[pl.ds(i * LANES, LANES)] = jnp.ones((LANES,), jnp.float32)
    return 0

  lax.fori_loop(0, CHUNK // LANES, _ones, 0)

  @pl.when(sid == 0)
  def _():
    lax.fori_loop(0, N // LANES, _zero, 0)
    pltpu.sync_copy(zbuf_v, deg_sh)

  plsc.subcore_barrier()

  base = wid * EDGES_PER_TILE

  def _chunk(c, _):
    pltpu.sync_copy(dst_hbm.at[pl.ds(base + c * CHUNK, CHUNK)], dst_v)
    pltpu.sync_copy(ones_v, deg_sh.at[dst_v], add=True)
    return 0

  lax.fori_loop(0, NCHUNK, _chunk, 0)

  plsc.subcore_barrier()

  @pl.when(sid == 0)
  def _():
    pltpu.sync_copy(deg_sh, zbuf_v)
    pltpu.sync_copy(zbuf_v, deg_hbm.at[pl.ds(cid * N, N)])


@functools.partial(
    pl.kernel,
    out_type=jax.ShapeDtypeStruct((NC, N, H), jnp.float32),
    mesh=_MESH,
    compiler_params=_SC_PARAMS,
    scratch_types=[
        [pltpu.VMEM((2 * CHUNK,), jnp.int32)] * 2,
        [pltpu.VMEM((CHUNK,), jnp.float32)] * 2,
        [pltpu.VMEM((CHUNK,), jnp.float32)] * 2,
        [pltpu.VMEM((CHUNK,), jnp.int32)] * 2,
        pltpu.VMEM((CHUNK,), jnp.float32),
        [pltpu.VMEM((CHUNK, H), jnp.float32)] * 2,
        [pltpu.VMEM((CHUNK, H), jnp.float32)] * 2,
        [pltpu.SemaphoreType.DMA] * 2,
        [pltpu.SemaphoreType.DMA] * 2,
        pltpu.VMEM_SHARED((N, H), jnp.float32),
    ],
)
def _sc_layer(g_hbm, al_hbm, ar_hbm, eidx_hbm, agg_hbm,
              ev, alv, arv, dvs, coef_v, ra, rs, semg, sems, agg_sh):
  cid = lax.axis_index("c")
  sid = lax.axis_index("s")
  wid = cid * NS + sid
  base = wid * NCHUNK

  def _zero(i, _):
    ra[0][i // (H // LANES), pl.ds((i % (H // LANES)) * LANES, LANES)] = (
        jnp.zeros((LANES,), jnp.float32))
    return 0

  lax.fori_loop(0, CHUNK * (H // LANES), _zero, 0)

  @pl.when(sid == 0)
  def _():
    def _zcopy(k, _):
      pltpu.sync_copy(ra[0], agg_sh.at[pl.ds(k * CHUNK, CHUNK)])
      return 0

    lax.fori_loop(0, N // CHUNK, _zcopy, 0)

  plsc.subcore_barrier()

  def _fetch(c, b):
    pltpu.sync_copy(eidx_hbm.at[pl.ds((base + c) * 2 * CHUNK, 2 * CHUNK)],
                    ev[b])
    pltpu.async_copy(g_hbm.at[ev[b].at[pl.ds(0, CHUNK)]], ra[b], semg[b])
    pltpu.async_copy(al_hbm.at[ev[b].at[pl.ds(0, CHUNK)]], alv[b], semg[b])
    pltpu.async_copy(ar_hbm.at[ev[b].at[pl.ds(CHUNK, CHUNK)]], arv[b],
                     semg[b])

  def _step(c, b, prefetch):
    pltpu.make_async_copy(g_hbm.at[pl.ds(0, CHUNK)], ra[b], semg[b]).wait()
    pltpu.make_async_copy(al_hbm.at[pl.ds(0, CHUNK)], alv[b], semg[b]).wait()
    pltpu.make_async_copy(al_hbm.at[pl.ds(0, CHUNK)], arv[b], semg[b]).wait()

    @pl.when(c >= 2)
    def _():
      pltpu.make_async_copy(g_hbm.at[pl.ds(0, CHUNK)], rs[b], sems[b]).wait()

    @plsc.parallel_loop(0, CHUNK // LANES, unroll=5)
    def _(g):
      sl = pl.ds(g * LANES, LANES)
      coef_v[sl] = _tanh(alv[b][sl] + arv[b][sl])
      dvs[b][sl] = ev[b][pl.ds(CHUNK + g * LANES, LANES)]

    @plsc.parallel_loop(0, CHUNK, unroll=4)
    def _(e):
      w = plsc.load_gather(coef_v, [jnp.full((LANES,), e, jnp.int32)])
      for j in range(H // LANES):
        sl = pl.ds(j * LANES, LANES)
        rs[b][e, sl] = ra[b][e, sl] * w

    pltpu.async_copy(rs[b], agg_sh.at[dvs[b]], sems[b], add=True)

    if prefetch:
      @pl.when(c + 2 < NCHUNK)
      def _():
        _fetch(c + 2, b)

  _fetch(0, 0)
  _fetch(1, 1)

  def _pair(k, _):
    _step(2 * k, 0, True)
    _step(2 * k + 1, 1, True)
    return 0

  lax.fori_loop(0, NCHUNK // 2, _pair, 0)
  _step(NCHUNK - 1, 0, False)

  pltpu.make_async_copy(g_hbm.at[pl.ds(0, CHUNK)], rs[1], sems[1]).wait()
  pltpu.make_async_copy(g_hbm.at[pl.ds(0, CHUNK)], rs[0], sems[0]).wait()

  plsc.subcore_barrier()

  @pl.when(sid == 0)
  def _():
    pltpu.sync_copy(agg_sh, agg_hbm.at[cid])


_BLK = 1000
_NBLK = N // _BLK
_PREC = jax.lax.Precision.HIGHEST


def _tc_prolog_body(x_ref, w1_ref, b1_ref, deg_ref, watt_ref, batt_ref,
                    h_ref, g_ref, alr_ref, dis_ref):
  h = lax.dot_general(x_ref[...], w1_ref[...], (((1,), (1,)), ((), ())),
                      precision=_PREC) + b1_ref[...]
  h = jnp.maximum(h, 0.0)
  h_ref[...] = h
  alr_ref[...] = lax.dot_general(h, watt_ref[...], (((1,), (0,)), ((), ())),
                                 precision=_PREC) + batt_ref[...]
  deg = deg_ref[...]
  d = deg[:, 0:1] + deg[:, 1:2]
  dis = jnp.where(d > 0.0, lax.rsqrt(jnp.where(d > 0.0, d, 1.0)), 0.0)
  dis_ref[...] = dis
  g_ref[...] = h * dis


def _tc_prolog(x, t1_w, b1, deg2t, watt, batt):
  return pl.pallas_call(
      _tc_prolog_body,
      grid=(_NBLK,),
      in_specs=[
          pl.BlockSpec((_BLK, D), lambda i: (i, 0)),
          pl.BlockSpec((H, D), lambda i: (0, 0)),
          pl.BlockSpec((1, H), lambda i: (0, 0)),
          pl.BlockSpec((_BLK, 2), lambda i: (i, 0)),
          pl.BlockSpec((H, 2), lambda i: (0, 0)),
          pl.BlockSpec((1, 2), lambda i: (0, 0)),
      ],
      out_specs=[
          pl.BlockSpec((_BLK, H), lambda i: (i, 0)),
          pl.BlockSpec((_BLK, H), lambda i: (i, 0)),
          pl.BlockSpec((_BLK, 2), lambda i: (i, 0)),
          pl.BlockSpec((_BLK, 1), lambda i: (i, 0)),
      ],
      out_shape=[
          jax.ShapeDtypeStruct((N, H), jnp.float32),
          jax.ShapeDtypeStruct((N, H), jnp.float32),
          jax.ShapeDtypeStruct((N, 2), jnp.float32),
          jax.ShapeDtypeStruct((N, 1), jnp.float32),
      ],
  )(x, t1_w, b1, deg2t, watt, batt)


def _tc_combine_body(agg_ref, raw_ref, dis_ref, watt_ref, batt_ref,
                     g_ref, alr_ref):
  dis = dis_ref[...]
  h = dis * (agg_ref[0] + agg_ref[1]) + EPS * raw_ref[...]
  g_ref[...] = h * dis
  alr_ref[...] = lax.dot_general(h, watt_ref[...], (((1,), (0,)), ((), ())),
                                 precision=_PREC) + batt_ref[...]


def _tc_combine(aggp, raw, dis2, watt, batt):
  return pl.pallas_call(
      _tc_combine_body,
      grid=(_NBLK,),
      in_specs=[
          pl.BlockSpec((NC, _BLK, H), lambda i: (0, i, 0)),
          pl.BlockSpec((_BLK, H), lambda i: (i, 0)),
          pl.BlockSpec((_BLK, 1), lambda i: (i, 0)),
          pl.BlockSpec((H, 2), lambda i: (0, 0)),
          pl.BlockSpec((1, 2), lambda i: (0, 0)),
      ],
      out_specs=[
          pl.BlockSpec((_BLK, H), lambda i: (i, 0)),
          pl.BlockSpec((_BLK, 2), lambda i: (i, 0)),
      ],
      out_shape=[
          jax.ShapeDtypeStruct((N, H), jnp.float32),
          jax.ShapeDtypeStruct((N, 2), jnp.float32),
      ],
  )(aggp, raw, dis2, watt, batt)


def _tc_epilog_body(agg_ref, raw_ref, dis_ref, w2_ref, b2_ref, batch_ref,
                    h_ref, gemb_ref):
  i = pl.program_id(0)
  h = dis_ref[...] * (agg_ref[0] + agg_ref[1]) + EPS * raw_ref[...]
  oh = lax.dot_general(h, w2_ref[...], (((1,), (1,)), ((), ())),
                       precision=_PREC) + b2_ref[...]
  h_ref[...] = oh
  gids = lax.broadcasted_iota(jnp.int32, (1, G), 1)
  onehot = (batch_ref[...] == gids).astype(jnp.float32)
  contrib = lax.dot_general(onehot, oh, (((0,), (0,)), ((), ())),
                            precision=_PREC)

  @pl.when(i == 0)
  def _():
    gemb_ref[...] = jnp.zeros_like(gemb_ref)

  gemb_ref[...] += contrib


def _tc_epilog(aggp, raw, dis2, t2_w, b2, batch2):
  return pl.pallas_call(
      _tc_epilog_body,
      grid=(_NBLK,),
      in_specs=[
          pl.BlockSpec((NC, _BLK, H), lambda i: (0, i, 0)),
          pl.BlockSpec((_BLK, H), lambda i: (i, 0)),
          pl.BlockSpec((_BLK, 1), lambda i: (i, 0)),
          pl.BlockSpec((H, H), lambda i: (0, 0)),
          pl.BlockSpec((1, H), lambda i: (0, 0)),
          pl.BlockSpec((_BLK, 1), lambda i: (i, 0)),
      ],
      out_specs=[
          pl.BlockSpec((_BLK, H), lambda i: (i, 0)),
          pl.BlockSpec((G, H), lambda i: (0, 0)),
      ],
      out_shape=[
          jax.ShapeDtypeStruct((N, H), jnp.float32),
          jax.ShapeDtypeStruct((G, H), jnp.float32),
      ],
  )(aggp, raw, dis2, t2_w, b2, batch2)


def kernel(x, edge_index, batch, t1_w, t1_b, t2_w, t2_b,
           att_l_w, att_l_b, att_r_w, att_r_b):
  src = edge_index[0]
  dst = edge_index[1]

  watts = [jnp.stack([att_l_w[l], att_r_w[l]], axis=1) for l in range(L)]
  batts = [jnp.stack([att_l_b[l], att_r_b[l]]).reshape(1, 2) for l in range(L)]
  b1 = t1_b.reshape(1, H)
  b2 = t2_b.reshape(1, H)
  batch2 = batch.reshape(N, 1)
  eidx = jnp.concatenate(
      [src.reshape(-1, CHUNK), dst.reshape(-1, CHUNK)], axis=1).reshape(-1)

  degp = _sc_degree(dst).reshape(NC, N)
  raw, g, alr, dis2 = _tc_prolog(x, t1_w, b1, degp.T, watts[0], batts[0])

  for l in range(L):
    aggp = _sc_layer(g, alr[:, 0], alr[:, 1], eidx)
    if l < L - 1:
      g, alr = _tc_combine(aggp, raw, dis2, watts[l + 1], batts[l + 1])

  out_h, graph_emb = _tc_epilog(aggp, raw, dis2, t2_w, b2, batch2)
  return (graph_emb, out_h)


# speedup vs baseline: 2.5569x; 1.2195x over previous
import functools

import jax
import jax.numpy as jnp
from jax import lax
from jax.experimental import pallas as pl
from jax.experimental.pallas import tpu as pltpu
from jax.experimental.pallas import tpu_sc as plsc

N = 10000
E = 320000
D = 128
H = 128
L = 4
G = 64
EPS = 0.1

NC = 2
NS = 16
LANES = 16
NW = NC * NS
EDGES_PER_TILE = E // NW
CHUNK = 80
NCHUNK = EDGES_PER_TILE // CHUNK
ROWS_PER_TILE = N // NS
ZROWS = 200

_MESH = plsc.VectorSubcoreMesh(core_axis_name="c", subcore_axis_name="s")
_SC_PARAMS = pltpu.CompilerParams(needs_layout_passes=False)


def _tanh(s):
  u = jnp.exp(-2.0 * jnp.abs(s))
  return jnp.sign(s) * (1.0 - u) / (1.0 + u)


@functools.partial(
    pl.kernel,
    out_type=jax.ShapeDtypeStruct((NC * N,), jnp.float32),
    mesh=_MESH,
    compiler_params=_SC_PARAMS,
    scratch_types=[
        pltpu.VMEM((CHUNK,), jnp.int32),
        pltpu.VMEM((CHUNK,), jnp.float32),
        pltpu.VMEM((N,), jnp.float32),
        pltpu.VMEM_SHARED((N,), jnp.float32),
    ],
)
def _sc_degree(dst_hbm, deg_hbm, dst_v, ones_v, zbuf_v, deg_sh):
  cid = lax.axis_index("c")
  sid = lax.axis_index("s")
  wid = cid * NS + sid

  def _zero(i, _):
    zbuf_v[pl.ds(i * LANES, LANES)] = jnp.zeros((LANES,), jnp.float32)
    return 0

  def _ones(i, _):
    ones_v[pl.ds(i * LANES, LANES)] = jnp.ones((LANES,), jnp.float32)
    return 0

  lax.fori_loop(0, CHUNK // LANES, _ones, 0)

  @pl.when(sid == 0)
  def _():
    lax.fori_loop(0, N // LANES, _zero, 0)
    pltpu.sync_copy(zbuf_v, deg_sh)

  plsc.subcore_barrier()

  base = wid * EDGES_PER_TILE

  def _chunk(c, _):
    pltpu.sync_copy(dst_hbm.at[pl.ds(base + c * CHUNK, CHUNK)], dst_v)
    pltpu.sync_copy(ones_v, deg_sh.at[dst_v], add=True)
    return 0

  lax.fori_loop(0, NCHUNK, _chunk, 0)

  plsc.subcore_barrier()

  @pl.when(sid == 0)
  def _():
    pltpu.sync_copy(deg_sh, zbuf_v)
    pltpu.sync_copy(zbuf_v, deg_hbm.at[pl.ds(cid * N, N)])


@functools.partial(
    pl.kernel,
    out_type=jax.ShapeDtypeStruct((NC, N, H), jnp.float32),
    mesh=_MESH,
    compiler_params=_SC_PARAMS,
    scratch_types=[
        [pltpu.VMEM((2 * CHUNK,), jnp.int32)] * 2,
        [pltpu.VMEM((CHUNK,), jnp.float32)] * 2,
        [pltpu.VMEM((CHUNK,), jnp.float32)] * 2,
        [pltpu.VMEM((CHUNK,), jnp.int32)] * 2,
        pltpu.VMEM((CHUNK,), jnp.float32),
        [pltpu.VMEM((CHUNK, H), jnp.float32)] * 2,
        [pltpu.VMEM((CHUNK, H), jnp.float32)] * 2,
        [pltpu.SemaphoreType.DMA] * 2,
        [pltpu.SemaphoreType.DMA] * 2,
        pltpu.VMEM_SHARED((N, H), jnp.float32),
    ],
)
def _sc_layer(g_hbm, al_hbm, ar_hbm, eidx_hbm, agg_hbm,
              ev, alv, arv, dvs, coef_v, ra, rs, semg, sems, agg_sh):
  cid = lax.axis_index("c")
  sid = lax.axis_index("s")
  wid = cid * NS + sid
  base = wid * NCHUNK

  def _zero(i, _):
    ra[0][i // (H // LANES), pl.ds((i % (H // LANES)) * LANES, LANES)] = (
        jnp.zeros((LANES,), jnp.float32))
    return 0

  lax.fori_loop(0, CHUNK * (H // LANES), _zero, 0)

  def _zcopy(k, _):
    blk = sid + NS * k

    @pl.when(blk < N // CHUNK)
    def _():
      pltpu.sync_copy(ra[0], agg_sh.at[pl.ds(blk * CHUNK, CHUNK)])

    return 0

  lax.fori_loop(0, (N // CHUNK + NS - 1) // NS, _zcopy, 0)

  plsc.subcore_barrier()

  def _fetch(c, b):
    pltpu.sync_copy(eidx_hbm.at[pl.ds((base + c) * 2 * CHUNK, 2 * CHUNK)],
                    ev[b])
    pltpu.async_copy(g_hbm.at[ev[b].at[pl.ds(0, CHUNK)]], ra[b], semg[b])
    pltpu.async_copy(al_hbm.at[ev[b].at[pl.ds(0, CHUNK)]], alv[b], semg[b])
    pltpu.async_copy(ar_hbm.at[ev[b].at[pl.ds(CHUNK, CHUNK)]], arv[b],
                     semg[b])

  def _step(c, b, prefetch):
    pltpu.make_async_copy(g_hbm.at[pl.ds(0, CHUNK)], ra[b], semg[b]).wait()
    pltpu.make_async_copy(al_hbm.at[pl.ds(0, CHUNK)], alv[b], semg[b]).wait()
    pltpu.make_async_copy(al_hbm.at[pl.ds(0, CHUNK)], arv[b], semg[b]).wait()

    @pl.when(c >= 2)
    def _():
      pltpu.make_async_copy(g_hbm.at[pl.ds(0, CHUNK)], rs[b], sems[b]).wait()

    @plsc.parallel_loop(0, CHUNK // LANES, unroll=5)
    def _(g):
      sl = pl.ds(g * LANES, LANES)
      coef_v[sl] = _tanh(alv[b][sl] + arv[b][sl])
      dvs[b][sl] = ev[b][pl.ds(CHUNK + g * LANES, LANES)]

    @plsc.parallel_loop(0, CHUNK, unroll=8)
    def _(e):
      w = plsc.load_gather(coef_v, [jnp.full((LANES,), e, jnp.int32)])
      for j in range(H // LANES):
        sl = pl.ds(j * LANES, LANES)
        rs[b][e, sl] = ra[b][e, sl] * w

    pltpu.async_copy(rs[b], agg_sh.at[dvs[b]], sems[b], add=True)

    if prefetch:
      @pl.when(c + 2 < NCHUNK)
      def _():
        _fetch(c + 2, b)

  _fetch(0, 0)
  _fetch(1, 1)

  def _pair(k, _):
    _step(2 * k, 0, True)
    _step(2 * k + 1, 1, True)
    return 0

  lax.fori_loop(0, NCHUNK // 2, _pair, 0)
  _step(NCHUNK - 1, 0, False)

  pltpu.make_async_copy(g_hbm.at[pl.ds(0, CHUNK)], rs[1], sems[1]).wait()
  pltpu.make_async_copy(g_hbm.at[pl.ds(0, CHUNK)], rs[0], sems[0]).wait()

  plsc.subcore_barrier()

  def _ocopy(k, _):
    blk = sid + NS * k

    @pl.when(blk < N // CHUNK)
    def _():
      pltpu.sync_copy(agg_sh.at[pl.ds(blk * CHUNK, CHUNK)],
                      agg_hbm.at[cid, pl.ds(blk * CHUNK, CHUNK)])

    return 0

  lax.fori_loop(0, (N // CHUNK + NS - 1) // NS, _ocopy, 0)


_BLK = 1000
_NBLK = N // _BLK
_PREC = jax.lax.Precision.HIGHEST


def _tc_prolog_body(x_ref, w1_ref, b1_ref, deg_ref, watt_ref, batt_ref,
                    h_ref, g_ref, alr_ref, dis_ref):
  h = lax.dot_general(x_ref[...], w1_ref[...], (((1,), (1,)), ((), ())),
                      precision=_PREC) + b1_ref[...]
  h = jnp.maximum(h, 0.0)
  h_ref[...] = h
  alr_ref[...] = lax.dot_general(h, watt_ref[...], (((1,), (0,)), ((), ())),
                                 precision=_PREC) + batt_ref[...]
  deg = deg_ref[...]
  d = deg[:, 0:1] + deg[:, 1:2]
  dis = jnp.where(d > 0.0, lax.rsqrt(jnp.where(d > 0.0, d, 1.0)), 0.0)
  dis_ref[...] = dis
  g_ref[...] = h * dis


def _tc_prolog(x, t1_w, b1, deg2t, watt, batt):
  return pl.pallas_call(
      _tc_prolog_body,
      grid=(_NBLK,),
      in_specs=[
          pl.BlockSpec((_BLK, D), lambda i: (i, 0)),
          pl.BlockSpec((H, D), lambda i: (0, 0)),
          pl.BlockSpec((1, H), lambda i: (0, 0)),
          pl.BlockSpec((_BLK, 2), lambda i: (i, 0)),
          pl.BlockSpec((H, 2), lambda i: (0, 0)),
          pl.BlockSpec((1, 2), lambda i: (0, 0)),
      ],
      out_specs=[
          pl.BlockSpec((_BLK, H), lambda i: (i, 0)),
          pl.BlockSpec((_BLK, H), lambda i: (i, 0)),
          pl.BlockSpec((_BLK, 2), lambda i: (i, 0)),
          pl.BlockSpec((_BLK, 1), lambda i: (i, 0)),
      ],
      out_shape=[
          jax.ShapeDtypeStruct((N, H), jnp.float32),
          jax.ShapeDtypeStruct((N, H), jnp.float32),
          jax.ShapeDtypeStruct((N, 2), jnp.float32),
          jax.ShapeDtypeStruct((N, 1), jnp.float32),
      ],
  )(x, t1_w, b1, deg2t, watt, batt)


def _tc_combine_body(agg_ref, raw_ref, dis_ref, watt_ref, batt_ref,
                     g_ref, alr_ref):
  dis = dis_ref[...]
  h = dis * (agg_ref[0] + agg_ref[1]) + EPS * raw_ref[...]
  g_ref[...] = h * dis
  alr_ref[...] = lax.dot_general(h, watt_ref[...], (((1,), (0,)), ((), ())),
                                 precision=_PREC) + batt_ref[...]


def _tc_combine(aggp, raw, dis2, watt, batt):
  return pl.pallas_call(
      _tc_combine_body,
      grid=(_NBLK,),
      in_specs=[
          pl.BlockSpec((NC, _BLK, H), lambda i: (0, i, 0)),
          pl.BlockSpec((_BLK, H), lambda i: (i, 0)),
          pl.BlockSpec((_BLK, 1), lambda i: (i, 0)),
          pl.BlockSpec((H, 2), lambda i: (0, 0)),
          pl.BlockSpec((1, 2), lambda i: (0, 0)),
      ],
      out_specs=[
          pl.BlockSpec((_BLK, H), lambda i: (i, 0)),
          pl.BlockSpec((_BLK, 2), lambda i: (i, 0)),
      ],
      out_shape=[
          jax.ShapeDtypeStruct((N, H), jnp.float32),
          jax.ShapeDtypeStruct((N, 2), jnp.float32),
      ],
  )(aggp, raw, dis2, watt, batt)


def _tc_epilog_body(agg_ref, raw_ref, dis_ref, w2_ref, b2_ref, batch_ref,
                    h_ref, gemb_ref):
  i = pl.program_id(0)
  h = dis_ref[...] * (agg_ref[0] + agg_ref[1]) + EPS * raw_ref[...]
  oh = lax.dot_general(h, w2_ref[...], (((1,), (1,)), ((), ())),
                       precision=_PREC) + b2_ref[...]
  h_ref[...] = oh
  gids = lax.broadcasted_iota(jnp.int32, (1, G), 1)
  onehot = (batch_ref[...] == gids).astype(jnp.float32)
  contrib = lax.dot_general(onehot, oh, (((0,), (0,)), ((), ())),
                            precision=_PREC)

  @pl.when(i == 0)
  def _():
    gemb_ref[...] = jnp.zeros_like(gemb_ref)

  gemb_ref[...] += contrib


def _tc_epilog(aggp, raw, dis2, t2_w, b2, batch2):
  return pl.pallas_call(
      _tc_epilog_body,
      grid=(_NBLK,),
      in_specs=[
          pl.BlockSpec((NC, _BLK, H), lambda i: (0, i, 0)),
          pl.BlockSpec((_BLK, H), lambda i: (i, 0)),
          pl.BlockSpec((_BLK, 1), lambda i: (i, 0)),
          pl.BlockSpec((H, H), lambda i: (0, 0)),
          pl.BlockSpec((1, H), lambda i: (0, 0)),
          pl.BlockSpec((_BLK, 1), lambda i: (i, 0)),
      ],
      out_specs=[
          pl.BlockSpec((_BLK, H), lambda i: (i, 0)),
          pl.BlockSpec((G, H), lambda i: (0, 0)),
      ],
      out_shape=[
          jax.ShapeDtypeStruct((N, H), jnp.float32),
          jax.ShapeDtypeStruct((G, H), jnp.float32),
      ],
  )(aggp, raw, dis2, t2_w, b2, batch2)


def kernel(x, edge_index, batch, t1_w, t1_b, t2_w, t2_b,
           att_l_w, att_l_b, att_r_w, att_r_b):
  src = edge_index[0]
  dst = edge_index[1]

  watts = [jnp.stack([att_l_w[l], att_r_w[l]], axis=1) for l in range(L)]
  batts = [jnp.stack([att_l_b[l], att_r_b[l]]).reshape(1, 2) for l in range(L)]
  b1 = t1_b.reshape(1, H)
  b2 = t2_b.reshape(1, H)
  batch2 = batch.reshape(N, 1)
  eidx = jnp.concatenate(
      [src.reshape(-1, CHUNK), dst.reshape(-1, CHUNK)], axis=1).reshape(-1)

  degp = _sc_degree(dst).reshape(NC, N)
  raw, g, alr, dis2 = _tc_prolog(x, t1_w, b1, degp.T, watts[0], batts[0])

  for l in range(L):
    aggp = _sc_layer(g, alr[:, 0], alr[:, 1], eidx)
    if l < L - 1:
      g, alr = _tc_combine(aggp, raw, dis2, watts[l + 1], batts[l + 1])

  out_h, graph_emb = _tc_epilog(aggp, raw, dis2, t2_w, b2, batch2)
  return (graph_emb, out_h)


# speedup vs baseline: 2.7150x; 1.0619x over previous
import functools

import jax
import jax.numpy as jnp
from jax import lax
from jax.experimental import pallas as pl
from jax.experimental.pallas import tpu as pltpu
from jax.experimental.pallas import tpu_sc as plsc

N = 10000
E = 320000
D = 128
H = 128
L = 4
G = 64
EPS = 0.1

NC = 2
NS = 16
LANES = 16
NW = NC * NS
EDGES_PER_TILE = E // NW
CHUNK = 80
NCHUNK = EDGES_PER_TILE // CHUNK
ROWS_PER_TILE = N // NS
ZROWS = 200

_MESH = plsc.VectorSubcoreMesh(core_axis_name="c", subcore_axis_name="s")
_SC_PARAMS = pltpu.CompilerParams(needs_layout_passes=False)


def _tanh(s):
  u = jnp.exp(-2.0 * jnp.abs(s))
  return jnp.sign(s) * (1.0 - u) / (1.0 + u)


@functools.partial(
    pl.kernel,
    out_type=jax.ShapeDtypeStruct((NC * N,), jnp.float32),
    mesh=_MESH,
    compiler_params=_SC_PARAMS,
    scratch_types=[
        pltpu.VMEM((NCHUNK, CHUNK), jnp.int32),
        pltpu.VMEM((CHUNK,), jnp.float32),
        pltpu.VMEM((N,), jnp.float32),
        pltpu.VMEM_SHARED((N,), jnp.float32),
    ],
)
def _sc_degree(dst_hbm, deg_hbm, dst_v, ones_v, zbuf_v, deg_sh):
  cid = lax.axis_index("c")
  sid = lax.axis_index("s")
  wid = cid * NS + sid

  def _zero(i, _):
    zbuf_v[pl.ds(i * LANES, LANES)] = jnp.zeros((LANES,), jnp.float32)
    return 0

  def _ones(i, _):
    ones_v[pl.ds(i * LANES, LANES)] = jnp.ones((LANES,), jnp.float32)
    return 0

  lax.fori_loop(0, CHUNK // LANES, _ones, 0)
  pltpu.sync_copy(dst_hbm.at[wid], dst_v)

  @pl.when(sid == 0)
  def _():
    lax.fori_loop(0, N // LANES, _zero, 0)
    pltpu.sync_copy(zbuf_v, deg_sh)

  plsc.subcore_barrier()

  def _chunk(c, _):
    pltpu.sync_copy(ones_v, deg_sh.at[dst_v.at[c]], add=True)
    return 0

  lax.fori_loop(0, NCHUNK, _chunk, 0)

  plsc.subcore_barrier()

  @pl.when(sid == 0)
  def _():
    pltpu.sync_copy(deg_sh, zbuf_v)
    pltpu.sync_copy(zbuf_v, deg_hbm.at[pl.ds(cid * N, N)])


@functools.partial(
    pl.kernel,
    out_type=jax.ShapeDtypeStruct((NC, N, H), jnp.float32),
    mesh=_MESH,
    compiler_params=_SC_PARAMS,
    scratch_types=[
        [pltpu.VMEM((2 * CHUNK,), jnp.int32)] * 2,
        [pltpu.VMEM((CHUNK,), jnp.float32)] * 2,
        [pltpu.VMEM((CHUNK,), jnp.float32)] * 2,
        [pltpu.VMEM((CHUNK,), jnp.int32)] * 2,
        pltpu.VMEM((CHUNK,), jnp.float32),
        [pltpu.VMEM((CHUNK, H), jnp.float32)] * 2,
        [pltpu.VMEM((CHUNK, H), jnp.float32)] * 2,
        [pltpu.SemaphoreType.DMA] * 2,
        [pltpu.SemaphoreType.DMA] * 2,
        pltpu.VMEM_SHARED((N, H), jnp.float32),
    ],
)
def _sc_layer(g_hbm, al_hbm, ar_hbm, eidx_hbm, agg_hbm,
              ev, alv, arv, dvs, coef_v, ra, rs, semg, sems, agg_sh):
  cid = lax.axis_index("c")
  sid = lax.axis_index("s")
  wid = cid * NS + sid
  base = wid * NCHUNK

  def _zero(i, _):
    ra[0][i // (H // LANES), pl.ds((i % (H // LANES)) * LANES, LANES)] = (
        jnp.zeros((LANES,), jnp.float32))
    return 0

  lax.fori_loop(0, CHUNK * (H // LANES), _zero, 0)

  def _zcopy(k, _):
    blk = sid + NS * k

    @pl.when(blk < N // CHUNK)
    def _():
      pltpu.sync_copy(ra[0], agg_sh.at[pl.ds(blk * CHUNK, CHUNK)])

    return 0

  lax.fori_loop(0, (N // CHUNK + NS - 1) // NS, _zcopy, 0)

  plsc.subcore_barrier()

  def _fetch(c, b):
    pltpu.sync_copy(eidx_hbm.at[pl.ds((base + c) * 2 * CHUNK, 2 * CHUNK)],
                    ev[b])
    pltpu.async_copy(g_hbm.at[ev[b].at[pl.ds(0, CHUNK)]], ra[b], semg[b])
    pltpu.async_copy(al_hbm.at[ev[b].at[pl.ds(0, CHUNK)]], alv[b], semg[b])
    pltpu.async_copy(ar_hbm.at[ev[b].at[pl.ds(CHUNK, CHUNK)]], arv[b],
                     semg[b])

  def _step(c, b, prefetch):
    pltpu.make_async_copy(g_hbm.at[pl.ds(0, CHUNK)], ra[b], semg[b]).wait()
    pltpu.make_async_copy(al_hbm.at[pl.ds(0, CHUNK)], alv[b], semg[b]).wait()
    pltpu.make_async_copy(al_hbm.at[pl.ds(0, CHUNK)], arv[b], semg[b]).wait()

    @pl.when(c >= 2)
    def _():
      pltpu.make_async_copy(g_hbm.at[pl.ds(0, CHUNK)], rs[b], sems[b]).wait()

    @plsc.parallel_loop(0, CHUNK // LANES, unroll=5)
    def _(g):
      sl = pl.ds(g * LANES, LANES)
      coef_v[sl] = _tanh(alv[b][sl] + arv[b][sl])
      dvs[b][sl] = ev[b][pl.ds(CHUNK + g * LANES, LANES)]

    @plsc.parallel_loop(0, CHUNK, unroll=16)
    def _(e):
      w = plsc.load_gather(coef_v, [jnp.full((LANES,), e, jnp.int32)])
      for j in range(H // LANES):
        sl = pl.ds(j * LANES, LANES)
        rs[b][e, sl] = ra[b][e, sl] * w

    pltpu.async_copy(rs[b], agg_sh.at[dvs[b]], sems[b], add=True)

    if prefetch:
      @pl.when(c + 2 < NCHUNK)
      def _():
        _fetch(c + 2, b)

  _fetch(0, 0)
  _fetch(1, 1)

  def _pair(k, _):
    _step(2 * k, 0, True)
    _step(2 * k + 1, 1, True)
    return 0

  lax.fori_loop(0, NCHUNK // 2, _pair, 0)
  _step(NCHUNK - 1, 0, False)

  pltpu.make_async_copy(g_hbm.at[pl.ds(0, CHUNK)], rs[1], sems[1]).wait()
  pltpu.make_async_copy(g_hbm.at[pl.ds(0, CHUNK)], rs[0], sems[0]).wait()

  plsc.subcore_barrier()

  def _ocopy(k, _):
    blk = sid + NS * k

    @pl.when(blk < N // CHUNK)
    def _():
      pltpu.sync_copy(agg_sh.at[pl.ds(blk * CHUNK, CHUNK)],
                      agg_hbm.at[cid, pl.ds(blk * CHUNK, CHUNK)])

    return 0

  lax.fori_loop(0, (N // CHUNK + NS - 1) // NS, _ocopy, 0)


_BLK = 1000
_NBLK = N // _BLK
_PREC = jax.lax.Precision.HIGHEST


def _tc_prolog_body(x_ref, w1_ref, b1_ref, deg_ref, watt_ref, batt_ref,
                    h_ref, g_ref, alr_ref, dis_ref):
  h = lax.dot_general(x_ref[...], w1_ref[...], (((1,), (1,)), ((), ())),
                      precision=_PREC) + b1_ref[...]
  h = jnp.maximum(h, 0.0)
  h_ref[...] = h
  alr_ref[...] = lax.dot_general(h, watt_ref[...], (((1,), (0,)), ((), ())),
                                 precision=_PREC) + batt_ref[...]
  deg = deg_ref[...]
  d = deg[:, 0:1] + deg[:, 1:2]
  dis = jnp.where(d > 0.0, lax.rsqrt(jnp.where(d > 0.0, d, 1.0)), 0.0)
  dis_ref[...] = dis
  g_ref[...] = h * dis


def _tc_prolog(x, t1_w, b1, deg2t, watt, batt):
  return pl.pallas_call(
      _tc_prolog_body,
      grid=(_NBLK,),
      in_specs=[
          pl.BlockSpec((_BLK, D), lambda i: (i, 0)),
          pl.BlockSpec((H, D), lambda i: (0, 0)),
          pl.BlockSpec((1, H), lambda i: (0, 0)),
          pl.BlockSpec((_BLK, 2), lambda i: (i, 0)),
          pl.BlockSpec((H, 2), lambda i: (0, 0)),
          pl.BlockSpec((1, 2), lambda i: (0, 0)),
      ],
      out_specs=[
          pl.BlockSpec((_BLK, H), lambda i: (i, 0)),
          pl.BlockSpec((_BLK, H), lambda i: (i, 0)),
          pl.BlockSpec((_BLK, 2), lambda i: (i, 0)),
          pl.BlockSpec((_BLK, 1), lambda i: (i, 0)),
      ],
      out_shape=[
          jax.ShapeDtypeStruct((N, H), jnp.float32),
          jax.ShapeDtypeStruct((N, H), jnp.float32),
          jax.ShapeDtypeStruct((N, 2), jnp.float32),
          jax.ShapeDtypeStruct((N, 1), jnp.float32),
      ],
  )(x, t1_w, b1, deg2t, watt, batt)


def _tc_combine_body(agg_ref, raw_ref, dis_ref, watt_ref, batt_ref,
                     g_ref, alr_ref):
  dis = dis_ref[...]
  h = dis * (agg_ref[0] + agg_ref[1]) + EPS * raw_ref[...]
  g_ref[...] = h * dis
  alr_ref[...] = lax.dot_general(h, watt_ref[...], (((1,), (0,)), ((), ())),
                                 precision=_PREC) + batt_ref[...]


def _tc_combine(aggp, raw, dis2, watt, batt):
  return pl.pallas_call(
      _tc_combine_body,
      grid=(_NBLK,),
      in_specs=[
          pl.BlockSpec((NC, _BLK, H), lambda i: (0, i, 0)),
          pl.BlockSpec((_BLK, H), lambda i: (i, 0)),
          pl.BlockSpec((_BLK, 1), lambda i: (i, 0)),
          pl.BlockSpec((H, 2), lambda i: (0, 0)),
          pl.BlockSpec((1, 2), lambda i: (0, 0)),
      ],
      out_specs=[
          pl.BlockSpec((_BLK, H), lambda i: (i, 0)),
          pl.BlockSpec((_BLK, 2), lambda i: (i, 0)),
      ],
      out_shape=[
          jax.ShapeDtypeStruct((N, H), jnp.float32),
          jax.ShapeDtypeStruct((N, 2), jnp.float32),
      ],
  )(aggp, raw, dis2, watt, batt)


def _tc_epilog_body(agg_ref, raw_ref, dis_ref, w2_ref, b2_ref, batch_ref,
                    h_ref, gemb_ref):
  i = pl.program_id(0)
  h = dis_ref[...] * (agg_ref[0] + agg_ref[1]) + EPS * raw_ref[...]
  oh = lax.dot_general(h, w2_ref[...], (((1,), (1,)), ((), ())),
                       precision=_PREC) + b2_ref[...]
  h_ref[...] = oh
  gids = lax.broadcasted_iota(jnp.int32, (1, G), 1)
  onehot = (batch_ref[...] == gids).astype(jnp.float32)
  contrib = lax.dot_general(onehot, oh, (((0,), (0,)), ((), ())),
                            precision=_PREC)

  @pl.when(i == 0)
  def _():
    gemb_ref[...] = jnp.zeros_like(gemb_ref)

  gemb_ref[...] += contrib


def _tc_epilog(aggp, raw, dis2, t2_w, b2, batch2):
  return pl.pallas_call(
      _tc_epilog_body,
      grid=(_NBLK,),
      in_specs=[
          pl.BlockSpec((NC, _BLK, H), lambda i: (0, i, 0)),
          pl.BlockSpec((_BLK, H), lambda i: (i, 0)),
          pl.BlockSpec((_BLK, 1), lambda i: (i, 0)),
          pl.BlockSpec((H, H), lambda i: (0, 0)),
          pl.BlockSpec((1, H), lambda i: (0, 0)),
          pl.BlockSpec((_BLK, 1), lambda i: (i, 0)),
      ],
      out_specs=[
          pl.BlockSpec((_BLK, H), lambda i: (i, 0)),
          pl.BlockSpec((G, H), lambda i: (0, 0)),
      ],
      out_shape=[
          jax.ShapeDtypeStruct((N, H), jnp.float32),
          jax.ShapeDtypeStruct((G, H), jnp.float32),
      ],
  )(aggp, raw, dis2, t2_w, b2, batch2)


def kernel(x, edge_index, batch, t1_w, t1_b, t2_w, t2_b,
           att_l_w, att_l_b, att_r_w, att_r_b):
  src = edge_index[0]
  dst = edge_index[1]

  watts = [jnp.stack([att_l_w[l], att_r_w[l]], axis=1) for l in range(L)]
  batts = [jnp.stack([att_l_b[l], att_r_b[l]]).reshape(1, 2) for l in range(L)]
  b1 = t1_b.reshape(1, H)
  b2 = t2_b.reshape(1, H)
  batch2 = batch.reshape(N, 1)
  eidx = jnp.concatenate(
      [src.reshape(-1, CHUNK), dst.reshape(-1, CHUNK)], axis=1).reshape(-1)

  dst3d = dst.reshape(NW, NCHUNK, CHUNK)
  degp = _sc_degree(dst3d).reshape(NC, N)
  raw, g, alr, dis2 = _tc_prolog(x, t1_w, b1, degp.T, watts[0], batts[0])

  for l in range(L):
    aggp = _sc_layer(g, alr[:, 0], alr[:, 1], eidx)
    if l < L - 1:
      g, alr = _tc_combine(aggp, raw, dis2, watts[l + 1], batts[l + 1])

  out_h, graph_emb = _tc_epilog(aggp, raw, dis2, t2_w, b2, batch2)
  return (graph_emb, out_h)


# speedup vs baseline: 2.7198x; 1.0018x over previous
import functools

import jax
import jax.numpy as jnp
from jax import lax
from jax.experimental import pallas as pl
from jax.experimental.pallas import tpu as pltpu
from jax.experimental.pallas import tpu_sc as plsc

N = 10000
E = 320000
D = 128
H = 128
L = 4
G = 64
EPS = 0.1

NC = 2
NS = 16
LANES = 16
NW = NC * NS
CHUNK = 80
NCHUNK = (E // NW) // CHUNK

_MESH = plsc.VectorSubcoreMesh(core_axis_name="c", subcore_axis_name="s")
_SC_PARAMS = pltpu.CompilerParams(needs_layout_passes=False)


def _tanh(s):
  u = jnp.exp(-2.0 * jnp.abs(s))
  return jnp.sign(s) * (1.0 - u) / (1.0 + u)


@functools.partial(
    pl.kernel,
    out_type=jax.ShapeDtypeStruct((NC * N,), jnp.float32),
    mesh=_MESH,
    compiler_params=_SC_PARAMS,
    scratch_types=[
        pltpu.VMEM((NCHUNK, CHUNK), jnp.int32),
        pltpu.VMEM((CHUNK,), jnp.float32),
        pltpu.VMEM((N,), jnp.float32),
        pltpu.VMEM_SHARED((N,), jnp.float32),
    ],
)
def _sc_degree(dst_hbm, deg_hbm, dst_v, ones_v, zbuf_v, deg_sh):
  cid = lax.axis_index("c")
  sid = lax.axis_index("s")
  wid = cid * NS + sid

  def _zero(i, _):
    zbuf_v[pl.ds(i * LANES, LANES)] = jnp.zeros((LANES,), jnp.float32)
    return 0

  def _ones(i, _):
    ones_v[pl.ds(i * LANES, LANES)] = jnp.ones((LANES,), jnp.float32)
    return 0

  lax.fori_loop(0, CHUNK // LANES, _ones, 0)
  pltpu.sync_copy(dst_hbm.at[wid], dst_v)

  @pl.when(sid == 0)
  def _():
    lax.fori_loop(0, N // LANES, _zero, 0)
    pltpu.sync_copy(zbuf_v, deg_sh)

  plsc.subcore_barrier()

  def _chunk(c, _):
    pltpu.sync_copy(ones_v, deg_sh.at[dst_v.at[c]], add=True)
    return 0

  lax.fori_loop(0, NCHUNK, _chunk, 0)

  plsc.subcore_barrier()

  @pl.when(sid == 0)
  def _():
    pltpu.sync_copy(deg_sh, zbuf_v)
    pltpu.sync_copy(zbuf_v, deg_hbm.at[pl.ds(cid * N, N)])


@functools.partial(
    pl.kernel,
    out_type=jax.ShapeDtypeStruct((NC, N, H), jnp.float32),
    mesh=_MESH,
    compiler_params=_SC_PARAMS,
    scratch_types=[
        [pltpu.VMEM((2 * CHUNK,), jnp.int32)] * 2,
        [pltpu.VMEM((CHUNK,), jnp.float32)] * 2,
        [pltpu.VMEM((CHUNK,), jnp.float32)] * 2,
        [pltpu.VMEM((CHUNK,), jnp.int32)] * 2,
        pltpu.VMEM((CHUNK,), jnp.float32),
        [pltpu.VMEM((CHUNK, H), jnp.float32)] * 2,
        [pltpu.VMEM((CHUNK, H), jnp.float32)] * 2,
        [pltpu.SemaphoreType.DMA] * 2,
        [pltpu.SemaphoreType.DMA] * 2,
        pltpu.VMEM_SHARED((N, H), jnp.float32),
    ],
)
def _sc_layer(g_hbm, al_hbm, ar_hbm, eidx_hbm, agg_hbm,
              ev, alv, arv, dvs, coef_v, ra, rs, semg, sems, agg_sh):
  cid = lax.axis_index("c")
  sid = lax.axis_index("s")
  wid = cid * NS + sid
  base = wid * NCHUNK

  def _zero(i, _):
    ra[0][i // (H // LANES), pl.ds((i % (H // LANES)) * LANES, LANES)] = (
        jnp.zeros((LANES,), jnp.float32))
    return 0

  lax.fori_loop(0, CHUNK * (H // LANES), _zero, 0)

  def _zcopy(k, _):
    blk = sid + NS * k

    @pl.when(blk < N // CHUNK)
    def _():
      pltpu.sync_copy(ra[0], agg_sh.at[pl.ds(blk * CHUNK, CHUNK)])

    return 0

  lax.fori_loop(0, (N // CHUNK + NS - 1) // NS, _zcopy, 0)

  plsc.subcore_barrier()

  def _fetch(c, b):
    pltpu.sync_copy(eidx_hbm.at[pl.ds((base + c) * 2 * CHUNK, 2 * CHUNK)],
                    ev[b])
    pltpu.async_copy(g_hbm.at[ev[b].at[pl.ds(0, CHUNK)]], ra[b], semg[b])
    pltpu.async_copy(al_hbm.at[ev[b].at[pl.ds(0, CHUNK)]], alv[b], semg[b])
    pltpu.async_copy(ar_hbm.at[ev[b].at[pl.ds(CHUNK, CHUNK)]], arv[b],
                     semg[b])

  def _step(c, b, prefetch):
    pltpu.make_async_copy(g_hbm.at[pl.ds(0, CHUNK)], ra[b], semg[b]).wait()
    pltpu.make_async_copy(al_hbm.at[pl.ds(0, CHUNK)], alv[b], semg[b]).wait()
    pltpu.make_async_copy(al_hbm.at[pl.ds(0, CHUNK)], arv[b], semg[b]).wait()

    @pl.when(c >= 2)
    def _():
      pltpu.make_async_copy(g_hbm.at[pl.ds(0, CHUNK)], rs[b], sems[b]).wait()

    @plsc.parallel_loop(0, CHUNK // LANES, unroll=5)
    def _(g):
      sl = pl.ds(g * LANES, LANES)
      coef_v[sl] = _tanh(alv[b][sl] + arv[b][sl])
      dvs[b][sl] = ev[b][pl.ds(CHUNK + g * LANES, LANES)]

    @plsc.parallel_loop(0, CHUNK, unroll=16)
    def _(e):
      w = plsc.load_gather(coef_v, [jnp.full((LANES,), e, jnp.int32)])
      for j in range(H // LANES):
        sl = pl.ds(j * LANES, LANES)
        rs[b][e, sl] = ra[b][e, sl] * w

    pltpu.async_copy(rs[b], agg_sh.at[dvs[b]], sems[b], add=True)

    if prefetch:
      @pl.when(c + 2 < NCHUNK)
      def _():
        _fetch(c + 2, b)

  _fetch(0, 0)
  _fetch(1, 1)

  def _pair(k, _):
    _step(2 * k, 0, True)
    _step(2 * k + 1, 1, True)
    return 0

  lax.fori_loop(0, NCHUNK // 2, _pair, 0)
  _step(NCHUNK - 1, 0, False)

  pltpu.make_async_copy(g_hbm.at[pl.ds(0, CHUNK)], rs[1], sems[1]).wait()
  pltpu.make_async_copy(g_hbm.at[pl.ds(0, CHUNK)], rs[0], sems[0]).wait()

  plsc.subcore_barrier()

  def _ocopy(k, _):
    blk = sid + NS * k

    @pl.when(blk < N // CHUNK)
    def _():
      pltpu.sync_copy(agg_sh.at[pl.ds(blk * CHUNK, CHUNK)],
                      agg_hbm.at[cid, pl.ds(blk * CHUNK, CHUNK)])

    return 0

  lax.fori_loop(0, (N // CHUNK + NS - 1) // NS, _ocopy, 0)


_BLK = 1000
_NBLK = N // _BLK
_PREC = jax.lax.Precision.HIGHEST


def _tc_prolog_body(x_ref, w1_ref, b1_ref, deg_ref, watt_ref, batt_ref,
                    h_ref, g_ref, alr_ref, dis_ref):
  h = lax.dot_general(x_ref[...], w1_ref[...], (((1,), (1,)), ((), ())),
                      precision=_PREC) + b1_ref[...]
  h = jnp.maximum(h, 0.0)
  h_ref[...] = h
  alr_ref[...] = lax.dot_general(h, watt_ref[...], (((1,), (0,)), ((), ())),
                                 precision=_PREC) + batt_ref[...]
  deg = deg_ref[...]
  d = deg[:, 0:1] + deg[:, 1:2]
  dis = jnp.where(d > 0.0, lax.rsqrt(jnp.where(d > 0.0, d, 1.0)), 0.0)
  dis_ref[...] = dis
  g_ref[...] = h * dis


def _tc_prolog(x, t1_w, b1, deg2t, watt, batt):
  return pl.pallas_call(
      _tc_prolog_body,
      grid=(_NBLK,),
      in_specs=[
          pl.BlockSpec((_BLK, D), lambda i: (i, 0)),
          pl.BlockSpec((H, D), lambda i: (0, 0)),
          pl.BlockSpec((1, H), lambda i: (0, 0)),
          pl.BlockSpec((_BLK, 2), lambda i: (i, 0)),
          pl.BlockSpec((H, 2), lambda i: (0, 0)),
          pl.BlockSpec((1, 2), lambda i: (0, 0)),
      ],
      out_specs=[
          pl.BlockSpec((_BLK, H), lambda i: (i, 0)),
          pl.BlockSpec((_BLK, H), lambda i: (i, 0)),
          pl.BlockSpec((_BLK, 2), lambda i: (i, 0)),
          pl.BlockSpec((_BLK, 1), lambda i: (i, 0)),
      ],
      out_shape=[
          jax.ShapeDtypeStruct((N, H), jnp.float32),
          jax.ShapeDtypeStruct((N, H), jnp.float32),
          jax.ShapeDtypeStruct((N, 2), jnp.float32),
          jax.ShapeDtypeStruct((N, 1), jnp.float32),
      ],
  )(x, t1_w, b1, deg2t, watt, batt)


def _tc_combine_body(agg_ref, raw_ref, dis_ref, watt_ref, batt_ref,
                     g_ref, alr_ref):
  dis = dis_ref[...]
  h = dis * (agg_ref[0] + agg_ref[1]) + EPS * raw_ref[...]
  g_ref[...] = h * dis
  alr_ref[...] = lax.dot_general(h, watt_ref[...], (((1,), (0,)), ((), ())),
                                 precision=_PREC) + batt_ref[...]


def _tc_combine(aggp, raw, dis2, watt, batt):
  return pl.pallas_call(
      _tc_combine_body,
      grid=(_NBLK,),
      in_specs=[
          pl.BlockSpec((NC, _BLK, H), lambda i: (0, i, 0)),
          pl.BlockSpec((_BLK, H), lambda i: (i, 0)),
          pl.BlockSpec((_BLK, 1), lambda i: (i, 0)),
          pl.BlockSpec((H, 2), lambda i: (0, 0)),
          pl.BlockSpec((1, 2), lambda i: (0, 0)),
      ],
      out_specs=[
          pl.BlockSpec((_BLK, H), lambda i: (i, 0)),
          pl.BlockSpec((_BLK, 2), lambda i: (i, 0)),
      ],
      out_shape=[
          jax.ShapeDtypeStruct((N, H), jnp.float32),
          jax.ShapeDtypeStruct((N, 2), jnp.float32),
      ],
  )(aggp, raw, dis2, watt, batt)


def _tc_epilog_body(agg_ref, raw_ref, dis_ref, w2_ref, b2_ref, batch_ref,
                    h_ref, gemb_ref):
  i = pl.program_id(0)
  h = dis_ref[...] * (agg_ref[0] + agg_ref[1]) + EPS * raw_ref[...]
  oh = lax.dot_general(h, w2_ref[...], (((1,), (1,)), ((), ())),
                       precision=_PREC) + b2_ref[...]
  h_ref[...] = oh
  gids = lax.broadcasted_iota(jnp.int32, (1, G), 1)
  onehot = (batch_ref[...] == gids).astype(jnp.float32)
  contrib = lax.dot_general(onehot, oh, (((0,), (0,)), ((), ())),
                            precision=_PREC)

  @pl.when(i == 0)
  def _():
    gemb_ref[...] = jnp.zeros_like(gemb_ref)

  gemb_ref[...] += contrib


def _tc_epilog(aggp, raw, dis2, t2_w, b2, batch2):
  return pl.pallas_call(
      _tc_epilog_body,
      grid=(_NBLK,),
      in_specs=[
          pl.BlockSpec((NC, _BLK, H), lambda i: (0, i, 0)),
          pl.BlockSpec((_BLK, H), lambda i: (i, 0)),
          pl.BlockSpec((_BLK, 1), lambda i: (i, 0)),
          pl.BlockSpec((H, H), lambda i: (0, 0)),
          pl.BlockSpec((1, H), lambda i: (0, 0)),
          pl.BlockSpec((_BLK, 1), lambda i: (i, 0)),
      ],
      out_specs=[
          pl.BlockSpec((_BLK, H), lambda i: (i, 0)),
          pl.BlockSpec((G, H), lambda i: (0, 0)),
      ],
      out_shape=[
          jax.ShapeDtypeStruct((N, H), jnp.float32),
          jax.ShapeDtypeStruct((G, H), jnp.float32),
      ],
  )(aggp, raw, dis2, t2_w, b2, batch2)


def kernel(x, edge_index, batch, t1_w, t1_b, t2_w, t2_b,
           att_l_w, att_l_b, att_r_w, att_r_b):
  src = edge_index[0]
  dst = edge_index[1]

  watts = [jnp.stack([att_l_w[l], att_r_w[l]], axis=1) for l in range(L)]
  batts = [jnp.stack([att_l_b[l], att_r_b[l]]).reshape(1, 2) for l in range(L)]
  b1 = t1_b.reshape(1, H)
  b2 = t2_b.reshape(1, H)
  batch2 = batch.reshape(N, 1)
  eidx = jnp.concatenate(
      [src.reshape(-1, CHUNK), dst.reshape(-1, CHUNK)], axis=1).reshape(-1)

  dst3d = dst.reshape(NW, NCHUNK, CHUNK)
  degp = _sc_degree(dst3d).reshape(NC, N)
  raw, g, alr, dis2 = _tc_prolog(x, t1_w, b1, degp.T, watts[0], batts[0])

  for l in range(L):
    aggp = _sc_layer(g, alr[:, 0], alr[:, 1], eidx)
    if l < L - 1:
      g, alr = _tc_combine(aggp, raw, dis2, watts[l + 1], batts[l + 1])

  out_h, graph_emb = _tc_epilog(aggp, raw, dis2, t2_w, b2, batch2)
  return (graph_emb, out_h)
